# baseline (speedup 1.0000x reference)
import functools
import math

import jax
import jax.numpy as jnp
from jax import lax
from jax.experimental import pallas as pl
from jax.experimental.pallas import tpu as pltpu
from jax.experimental.pallas import tpu_sc as plsc

_B, _S = 2, 2048
_DE = 128
_H = 4
_DH = _DE // _H
_E = 6
_HID = 128
_D = 3 * _DE
_C = 6

_TS = 512
_QB = 1024

_NEG = -1e30


def _f32dot(a, b):
    return jnp.dot(a, b, preferred_element_type=jnp.float32)


def _ln_block(x, g, b):
    m = jnp.mean(x, axis=-1, keepdims=True)
    d = x - m
    var = jnp.mean(d * d, axis=-1, keepdims=True)
    return d * jax.lax.rsqrt(var + 1e-5) * g + b


def _qkv_of(x, g, b, wqkv, bqkv):
    y = _ln_block(x, g, b)
    return _f32dot(y, wqkv) + bqkv


def _inproj_kernel(a_ref, t_ref, v_ref, wa, ba, wt, bt, wv, bv,
                   g1, b1, wqkv, bqkv, x_ref, q_ref, kv_ref):
    ins = ((a_ref, wa, ba), (t_ref, wt, bt), (v_ref, wv, bv))
    for m, (r, w, bb) in enumerate(ins):
        x = _f32dot(r[0], w[...]) + bb[...]
        x_ref[m, 0] = x
        qkv = _qkv_of(x, g1[...], b1[...], wqkv[...], bqkv[...])
        q_ref[m, 0] = qkv[:, :_DE]
        kv_ref[m, 0] = qkv[:, _DE:]


def _attn_kernel(q_ref, kv_ref, x_ref, wo, bo, g2, b2, wr_ref, br_ref,
                 h_ref, lg_ref):
    q_all = q_ref[0, 0]
    kv = kv_ref[0, 0]
    c = 1.4426950408889634 / math.sqrt(float(_DH))
    outs = []
    for h in range(_H):
        lo = h * _DH
        q = (q_all[:, lo:lo + _DH] * c).astype(jnp.bfloat16)
        k = kv[:, lo:lo + _DH].astype(jnp.bfloat16)
        v = kv[:, _DE + lo:_DE + lo + _DH]
        s = jax.lax.dot_general(q, k, (((1,), (1,)), ((), ())),
                                preferred_element_type=jnp.float32)
        p = jnp.exp2(s.astype(jnp.bfloat16))
        r = 1.0 / jnp.sum(p.astype(jnp.float32), axis=-1, keepdims=True)
        outs.append(jnp.dot(p, v.astype(jnp.bfloat16),
                            preferred_element_type=jnp.float32) * r)
    o = jnp.concatenate(outs, axis=-1)
    hh = x_ref[0, 0] + _f32dot(o, wo[...]) + bo[...]
    h_ref[0, 0] = hh
    z = _ln_block(hh, g2[...], b2[...])
    lt = jax.lax.dot_general(wr_ref[0], z, (((0,), (1,)), ((), ())),
                             preferred_element_type=jnp.float32)
    lg_ref[0, 0] = lt + br_ref[0]


def _moe_block(h, g2, b2, wts_t, w1all, b1all, w2all, b2mat, expand):
    z = _ln_block(h, g2, b2)
    hidden = jax.nn.gelu(_f32dot(z, w1all[...]) + b1all[...])
    wexp = jax.lax.dot_general(wts_t, expand[...], (((0,), (0,)), ((), ())),
                               preferred_element_type=jnp.float32)
    b2term = jax.lax.dot_general(wts_t, b2mat[...], (((0,), (0,)), ((), ())),
                                 preferred_element_type=jnp.float32)
    return h + _f32dot(wexp * hidden, w2all[...]) + b2term


def _sc_routing(logits):
    ginfo = plsc.get_sparse_core_info()
    nw = ginfo.num_cores * ginfo.num_subcores
    lanes = ginfo.num_lanes
    g_, e_, s_ = logits.shape
    chunk = s_ // nw

    mesh = plsc.VectorSubcoreMesh(core_axis_name="c", subcore_axis_name="s")

    @functools.partial(
        pl.kernel, mesh=mesh,
        out_type=jax.ShapeDtypeStruct((g_, e_, s_), jnp.float32),
        scratch_types=[pltpu.VMEM((e_, chunk), jnp.float32),
                       pltpu.VMEM((e_, chunk), jnp.float32)],
    )
    def k(lg_hbm, out_hbm, lg_v, wt_v):
        wid = lax.axis_index("s") * ginfo.num_cores + lax.axis_index("c")
        base = wid * chunk
        for g in range(g_):
            for e in range(e_):
                pltpu.sync_copy(lg_hbm.at[g, e, pl.ds(base, chunk)],
                                lg_v.at[e])
            for cidx in range(chunk // lanes):
                sl = pl.ds(cidx * lanes, lanes)
                l = [lg_v[e, sl] for e in range(e_)]
                m1 = l[0]
                for e in range(1, e_):
                    m1 = jnp.maximum(m1, l[e])
                taken = jnp.zeros_like(m1)
                msk1 = []
                for e in range(e_):
                    hit = jnp.where(l[e] == m1, 1.0 - taken, 0.0)
                    msk1.append(hit)
                    taken = taken + hit
                rest = [jnp.where(msk1[e] > 0.5, _NEG, l[e])
                        for e in range(e_)]
                m2 = rest[0]
                for e in range(1, e_):
                    m2 = jnp.maximum(m2, rest[e])
                g1w = 1.0 / (1.0 + jnp.exp(m2 - m1))
                g2w = 1.0 - g1w
                taken2 = jnp.zeros_like(m1)
                for e in range(e_):
                    hit2 = jnp.where(rest[e] == m2, 1.0 - taken2, 0.0)
                    taken2 = taken2 + hit2
                    wt_v[e, sl] = msk1[e] * g1w + hit2 * g2w
            for e in range(e_):
                pltpu.sync_copy(wt_v.at[e],
                                out_hbm.at[g, e, pl.ds(base, chunk)])

    return k(logits)


def _mid_kernel(h_ref, wts_ref, g2, b2,
                w1s, b1s, w2s, b2s, expand, g1n, b1n, wqkvn, bqkvn,
                xn_ref, qn_ref, kvn_ref):
    h = h_ref[0, 0]
    acc = _moe_block(h, g2[...], b2[...], wts_ref[0, 0],
                     w1s, b1s, w2s, b2s, expand)
    xn_ref[0, 0] = acc
    qkv = _qkv_of(acc, g1n[...], b1n[...], wqkvn[...], bqkvn[...])
    qn_ref[0, 0] = qkv[:, :_DE]
    kvn_ref[0, 0] = qkv[:, _DE:]


def _fin_kernel(h_ref, wts_ref, g2, b2,
                w1s, b1s, w2s, b2s, expand, wp1, bp1, wh, bh, o_ref):
    parts = []
    for m in range(3):
        parts.append(_moe_block(h_ref[m, 0], g2[...], b2[...], wts_ref[m, 0],
                                w1s, b1s, w2s, b2s, expand))
    fused = jnp.concatenate(parts, axis=-1)
    hid = jnp.maximum(_f32dot(fused, wp1[...]) + bp1[...], 0.0)
    o_ref[0] = _f32dot(hid, wh[...]) + bh[...]


def _full(shape):
    n = len(shape)
    return pl.BlockSpec(shape, lambda *args: (0,) * n)


def kernel(a, t, v, Wa, ba, Wt, bt, Wv, bv, ln1_g, ln1_b, Wqkv, bqkv, Wo, bo,
           ln2_g, ln2_b, Wr, br, W1, b1, W2, b2, Wp1, bp1, Wh, bh):
    f32 = jnp.float32
    r2 = lambda x: x.reshape(1, -1)
    expand = jnp.kron(jnp.eye(_E, dtype=f32), jnp.ones((1, _HID), f32))

    nst = _S // _TS
    nqb = _S // _QB

    tok = lambda w: pl.BlockSpec((1, _TS, w), lambda bb, si: (bb, si, 0))
    qkv_outspecs = [
        pl.BlockSpec((3, 1, _TS, _DE), lambda bb, si: (0, bb, si, 0)),
        pl.BlockSpec((3, 1, _TS, _DE), lambda bb, si: (0, bb, si, 0)),
        pl.BlockSpec((3, 1, _TS, 2 * _DE), lambda bb, si: (0, bb, si, 0))]
    qkv_outshapes = [jax.ShapeDtypeStruct((3, _B, _S, _DE), f32),
                     jax.ShapeDtypeStruct((3, _B, _S, _DE), f32),
                     jax.ShapeDtypeStruct((3, _B, _S, 2 * _DE), f32)]
    x0, q0, kv0 = pl.pallas_call(
        _inproj_kernel,
        grid=(_B, nst),
        in_specs=[tok(a.shape[-1]), tok(t.shape[-1]), tok(v.shape[-1])]
                 + [_full(s) for s in ((Wa.shape), (1, _DE), (Wt.shape), (1, _DE),
                                       (Wv.shape), (1, _DE), (1, _DE), (1, _DE),
                                       (_DE, 3 * _DE), (1, 3 * _DE))],
        out_specs=qkv_outspecs,
        out_shape=qkv_outshapes,
    )(a, t, v, Wa, r2(ba), Wt, r2(bt), Wv, r2(bv),
      r2(ln1_g[0]), r2(ln1_b[0]), Wqkv[0], r2(bqkv[0]))

    tokq = pl.BlockSpec((1, 1, _QB, _DE), lambda m, bb, si: (m, bb, si, 0))

    def attention(q, kv, x, l):
        h, lg = pl.pallas_call(
            _attn_kernel,
            grid=(3, _B, nqb),
            in_specs=[tokq,
                      pl.BlockSpec((1, 1, _S, 2 * _DE), lambda m, bb, si: (m, bb, 0, 0)),
                      tokq,
                      _full((_DE, _DE)), _full((1, _DE)),
                      _full((1, _DE)), _full((1, _DE)),
                      pl.BlockSpec((1, _DE, _E), lambda m, bb, si: (m, 0, 0)),
                      pl.BlockSpec((1, _E, 1), lambda m, bb, si: (m, 0, 0))],
            out_specs=[tokq,
                       pl.BlockSpec((1, 1, _E, _QB), lambda m, bb, si: (m, bb, 0, si))],
            out_shape=[jax.ShapeDtypeStruct((3, _B, _S, _DE), f32),
                       jax.ShapeDtypeStruct((3, _B, _E, _S), f32)],
        )(q, kv, x, Wo[l], r2(bo[l]), r2(ln2_g[l]), r2(ln2_b[l]),
          Wr[l], br[l].reshape(3, _E, 1))
        wts = _sc_routing(lg.reshape(3 * _B, _E, _S)).reshape(3, _B, _E, _S)
        return h, wts

    h0, wts0 = attention(q0, kv0, x0, 0)

    tokde = pl.BlockSpec((1, 1, _TS, _DE), lambda m, bb, si: (m, bb, si, 0))
    wspec = pl.BlockSpec((1, 1, _E, _TS), lambda m, bb, si: (m, bb, 0, si))
    x1, q1, kv1 = pl.pallas_call(
        _mid_kernel,
        grid=(3, _B, nst),
        in_specs=[tokde, wspec,
                  _full((1, _DE)), _full((1, _DE)),
                  _full((_DE, _E * _HID)), _full((1, _E * _HID)),
                  _full((_E * _HID, _DE)), _full((_E, _DE)),
                  _full((_E, _E * _HID)),
                  _full((1, _DE)), _full((1, _DE)),
                  _full((_DE, 3 * _DE)), _full((1, 3 * _DE))],
        out_specs=[tokde, tokde,
                   pl.BlockSpec((1, 1, _TS, 2 * _DE), lambda m, bb, si: (m, bb, si, 0))],
        out_shape=[jax.ShapeDtypeStruct((3, _B, _S, _DE), f32),
                   jax.ShapeDtypeStruct((3, _B, _S, _DE), f32),
                   jax.ShapeDtypeStruct((3, _B, _S, 2 * _DE), f32)],
    )(h0, wts0, r2(ln2_g[0]), r2(ln2_b[0]),
      W1[0].transpose(1, 0, 2).reshape(_DE, _E * _HID),
      b1[0].reshape(1, _E * _HID),
      W2[0].reshape(_E * _HID, _DE), b2[0], expand,
      r2(ln1_g[1]), r2(ln1_b[1]), Wqkv[1], r2(bqkv[1]))

    h1, wts1 = attention(q1, kv1, x1, 1)

    tok3 = pl.BlockSpec((3, 1, _TS, _DE), lambda bb, si: (0, bb, si, 0))
    wspec3 = pl.BlockSpec((3, 1, _E, _TS), lambda bb, si: (0, bb, 0, si))
    out = pl.pallas_call(
        _fin_kernel,
        grid=(_B, nst),
        in_specs=[tok3, wspec3,
                  _full((1, _DE)), _full((1, _DE)),
                  _full((_DE, _E * _HID)), _full((1, _E * _HID)),
                  _full((_E * _HID, _DE)), _full((_E, _DE)),
                  _full((_E, _E * _HID)),
                  _full((_D, _D)), _full((1, _D)),
                  _full((_D, _C)), _full((1, _C))],
        out_specs=pl.BlockSpec((1, _TS, _C), lambda bb, si: (bb, si, 0)),
        out_shape=jax.ShapeDtypeStruct((_B, _S, _C), f32),
    )(h1, wts1, r2(ln2_g[1]), r2(ln2_b[1]),
      W1[1].transpose(1, 0, 2).reshape(_DE, _E * _HID),
      b1[1].reshape(1, _E * _HID),
      W2[1].reshape(_E * _HID, _DE), b2[1], expand,
      Wp1, r2(bp1), Wh, r2(bh))
    return out


# speedup vs baseline: 1.0699x; 1.0699x over previous
import functools
import math

import jax
import jax.numpy as jnp
from jax import lax
from jax.experimental import pallas as pl
from jax.experimental.pallas import tpu as pltpu
from jax.experimental.pallas import tpu_sc as plsc

_B, _S = 2, 2048
_DE = 128
_H = 4
_DH = _DE // _H
_E = 6
_HID = 128
_D = 3 * _DE
_C = 6

_TS = 512
_QB = 1024

_NEG = -1e30


def _f32dot(a, b):
    return jnp.dot(a, b, preferred_element_type=jnp.float32)


def _ln_block(x, g, b):
    m = jnp.mean(x, axis=-1, keepdims=True)
    d = x - m
    var = jnp.mean(d * d, axis=-1, keepdims=True)
    return d * jax.lax.rsqrt(var + 1e-5) * g + b


def _qkv_of(x, g, b, wqkv, bqkv):
    y = _ln_block(x, g, b)
    return _f32dot(y, wqkv) + bqkv


def _inproj_kernel(a_ref, t_ref, v_ref, wa, ba, wt, bt, wv, bv,
                   g1, b1, wqkv, bqkv, x_ref, q_ref, kv_ref):
    ins = ((a_ref, wa, ba), (t_ref, wt, bt), (v_ref, wv, bv))
    for m, (r, w, bb) in enumerate(ins):
        x = _f32dot(r[0], w[...]) + bb[...]
        x_ref[m, 0] = x
        qkv = _qkv_of(x, g1[...], b1[...], wqkv[...], bqkv[...])
        q_ref[m, 0] = qkv[:, :_DE]
        kv_ref[m, 0] = qkv[:, _DE:]


def _attn_kernel(q_ref, kv_ref, x_ref, wo, bo, g2, b2, wr_ref, br_ref,
                 h_ref, lg_ref):
    q_all = q_ref[0, 0]
    kv = kv_ref[0, 0]
    c = 1.4426950408889634 / math.sqrt(float(_DH))
    outs = []
    for h in range(_H):
        lo = h * _DH
        q = (q_all[:, lo:lo + _DH] * c).astype(jnp.bfloat16)
        k = kv[:, lo:lo + _DH].astype(jnp.bfloat16)
        v = kv[:, _DE + lo:_DE + lo + _DH]
        s = jax.lax.dot_general(q, k, (((1,), (1,)), ((), ())),
                                preferred_element_type=jnp.float32)
        p = jnp.exp2(s.astype(jnp.bfloat16))
        r = 1.0 / jnp.sum(p.astype(jnp.float32), axis=-1, keepdims=True)
        outs.append(jnp.dot(p, v.astype(jnp.bfloat16),
                            preferred_element_type=jnp.float32) * r)
    o = jnp.concatenate(outs, axis=-1)
    hh = x_ref[0, 0] + _f32dot(o, wo[...]) + bo[...]
    h_ref[0, 0] = hh
    z = _ln_block(hh, g2[...], b2[...])
    lt = jax.lax.dot_general(wr_ref[0], z, (((0,), (1,)), ((), ())),
                             preferred_element_type=jnp.float32)
    lg_ref[0, 0] = lt + br_ref[0]


def _moe_block(h, g2, b2, wts_t, w1all, b1all, w2all, b2mat, expand):
    z = _ln_block(h, g2, b2)
    hidden = jax.nn.gelu(_f32dot(z, w1all[...]) + b1all[...])
    wexp = jax.lax.dot_general(wts_t, expand[...], (((0,), (0,)), ((), ())),
                               preferred_element_type=jnp.float32)
    b2term = jax.lax.dot_general(wts_t, b2mat[...], (((0,), (0,)), ((), ())),
                                 preferred_element_type=jnp.float32)
    return h + _f32dot(wexp * hidden, w2all[...]) + b2term


def _sc_routing(logits):
    ginfo = plsc.get_sparse_core_info()
    nw = ginfo.num_cores * ginfo.num_subcores
    lanes = ginfo.num_lanes
    g_, e_, s_ = logits.shape
    chunk = s_ // nw

    mesh = plsc.VectorSubcoreMesh(core_axis_name="c", subcore_axis_name="s")

    @functools.partial(
        pl.kernel, mesh=mesh,
        out_type=jax.ShapeDtypeStruct((g_, e_, s_), jnp.float32),
        scratch_types=[pltpu.VMEM((e_, chunk), jnp.float32),
                       pltpu.VMEM((e_, chunk), jnp.float32),
                       pltpu.SemaphoreType.DMA],
    )
    def k(lg_hbm, out_hbm, lg_v, wt_v, sem):
        wid = lax.axis_index("s") * ginfo.num_cores + lax.axis_index("c")
        base = wid * chunk
        for g in range(g_):
            cps = [pltpu.async_copy(lg_hbm.at[g, e, pl.ds(base, chunk)],
                                    lg_v.at[e], sem) for e in range(e_)]
            for cp in cps:
                cp.wait()
            for cidx in range(chunk // lanes):
                sl = pl.ds(cidx * lanes, lanes)
                l = [lg_v[e, sl] for e in range(e_)]
                m1 = l[0]
                for e in range(1, e_):
                    m1 = jnp.maximum(m1, l[e])
                taken = jnp.zeros_like(m1)
                msk1 = []
                for e in range(e_):
                    hit = jnp.where(l[e] == m1, 1.0 - taken, 0.0)
                    msk1.append(hit)
                    taken = taken + hit
                rest = [jnp.where(msk1[e] > 0.5, _NEG, l[e])
                        for e in range(e_)]
                m2 = rest[0]
                for e in range(1, e_):
                    m2 = jnp.maximum(m2, rest[e])
                g1w = 1.0 / (1.0 + jnp.exp(m2 - m1))
                g2w = 1.0 - g1w
                taken2 = jnp.zeros_like(m1)
                for e in range(e_):
                    hit2 = jnp.where(rest[e] == m2, 1.0 - taken2, 0.0)
                    taken2 = taken2 + hit2
                    wt_v[e, sl] = msk1[e] * g1w + hit2 * g2w
            ops = [pltpu.async_copy(wt_v.at[e],
                                    out_hbm.at[g, e, pl.ds(base, chunk)],
                                    sem) for e in range(e_)]
            for cp in ops:
                cp.wait()

    return k(logits)


def _mid_kernel(h_ref, wts_ref, g2, b2,
                w1s, b1s, w2s, b2s, expand, g1n, b1n, wqkvn, bqkvn,
                xn_ref, qn_ref, kvn_ref):
    h = h_ref[0, 0]
    acc = _moe_block(h, g2[...], b2[...], wts_ref[0, 0],
                     w1s, b1s, w2s, b2s, expand)
    xn_ref[0, 0] = acc
    qkv = _qkv_of(acc, g1n[...], b1n[...], wqkvn[...], bqkvn[...])
    qn_ref[0, 0] = qkv[:, :_DE]
    kvn_ref[0, 0] = qkv[:, _DE:]


def _fin_kernel(h_ref, wts_ref, g2, b2,
                w1s, b1s, w2s, b2s, expand, wp1, bp1, wh, bh, o_ref):
    parts = []
    for m in range(3):
        parts.append(_moe_block(h_ref[m, 0], g2[...], b2[...], wts_ref[m, 0],
                                w1s, b1s, w2s, b2s, expand))
    fused = jnp.concatenate(parts, axis=-1)
    hid = jnp.maximum(_f32dot(fused, wp1[...]) + bp1[...], 0.0)
    o_ref[0] = _f32dot(hid, wh[...]) + bh[...]


def _full(shape):
    n = len(shape)
    return pl.BlockSpec(shape, lambda *args: (0,) * n)


def kernel(a, t, v, Wa, ba, Wt, bt, Wv, bv, ln1_g, ln1_b, Wqkv, bqkv, Wo, bo,
           ln2_g, ln2_b, Wr, br, W1, b1, W2, b2, Wp1, bp1, Wh, bh):
    f32 = jnp.float32
    r2 = lambda x: x.reshape(1, -1)
    expand = jnp.kron(jnp.eye(_E, dtype=f32), jnp.ones((1, _HID), f32))

    nst = _S // _TS
    nqb = _S // _QB

    tok = lambda w: pl.BlockSpec((1, _TS, w), lambda bb, si: (bb, si, 0))
    qkv_outspecs = [
        pl.BlockSpec((3, 1, _TS, _DE), lambda bb, si: (0, bb, si, 0)),
        pl.BlockSpec((3, 1, _TS, _DE), lambda bb, si: (0, bb, si, 0)),
        pl.BlockSpec((3, 1, _TS, 2 * _DE), lambda bb, si: (0, bb, si, 0))]
    qkv_outshapes = [jax.ShapeDtypeStruct((3, _B, _S, _DE), f32),
                     jax.ShapeDtypeStruct((3, _B, _S, _DE), f32),
                     jax.ShapeDtypeStruct((3, _B, _S, 2 * _DE), f32)]
    x0, q0, kv0 = pl.pallas_call(
        _inproj_kernel,
        grid=(_B, nst),
        in_specs=[tok(a.shape[-1]), tok(t.shape[-1]), tok(v.shape[-1])]
                 + [_full(s) for s in ((Wa.shape), (1, _DE), (Wt.shape), (1, _DE),
                                       (Wv.shape), (1, _DE), (1, _DE), (1, _DE),
                                       (_DE, 3 * _DE), (1, 3 * _DE))],
        out_specs=qkv_outspecs,
        out_shape=qkv_outshapes,
    )(a, t, v, Wa, r2(ba), Wt, r2(bt), Wv, r2(bv),
      r2(ln1_g[0]), r2(ln1_b[0]), Wqkv[0], r2(bqkv[0]))

    tokq = pl.BlockSpec((1, 1, _QB, _DE), lambda m, bb, si: (m, bb, si, 0))

    def attention(q, kv, x, l):
        h, lg = pl.pallas_call(
            _attn_kernel,
            grid=(3, _B, nqb),
            in_specs=[tokq,
                      pl.BlockSpec((1, 1, _S, 2 * _DE), lambda m, bb, si: (m, bb, 0, 0)),
                      tokq,
                      _full((_DE, _DE)), _full((1, _DE)),
                      _full((1, _DE)), _full((1, _DE)),
                      pl.BlockSpec((1, _DE, _E), lambda m, bb, si: (m, 0, 0)),
                      pl.BlockSpec((1, _E, 1), lambda m, bb, si: (m, 0, 0))],
            out_specs=[tokq,
                       pl.BlockSpec((1, 1, _E, _QB), lambda m, bb, si: (m, bb, 0, si))],
            out_shape=[jax.ShapeDtypeStruct((3, _B, _S, _DE), f32),
                       jax.ShapeDtypeStruct((3, _B, _E, _S), f32)],
        )(q, kv, x, Wo[l], r2(bo[l]), r2(ln2_g[l]), r2(ln2_b[l]),
          Wr[l], br[l].reshape(3, _E, 1))
        wts = _sc_routing(lg.reshape(3 * _B, _E, _S)).reshape(3, _B, _E, _S)
        return h, wts

    h0, wts0 = attention(q0, kv0, x0, 0)

    tokde = pl.BlockSpec((1, 1, _TS, _DE), lambda m, bb, si: (m, bb, si, 0))
    wspec = pl.BlockSpec((1, 1, _E, _TS), lambda m, bb, si: (m, bb, 0, si))
    x1, q1, kv1 = pl.pallas_call(
        _mid_kernel,
        grid=(3, _B, nst),
        in_specs=[tokde, wspec,
                  _full((1, _DE)), _full((1, _DE)),
                  _full((_DE, _E * _HID)), _full((1, _E * _HID)),
                  _full((_E * _HID, _DE)), _full((_E, _DE)),
                  _full((_E, _E * _HID)),
                  _full((1, _DE)), _full((1, _DE)),
                  _full((_DE, 3 * _DE)), _full((1, 3 * _DE))],
        out_specs=[tokde, tokde,
                   pl.BlockSpec((1, 1, _TS, 2 * _DE), lambda m, bb, si: (m, bb, si, 0))],
        out_shape=[jax.ShapeDtypeStruct((3, _B, _S, _DE), f32),
                   jax.ShapeDtypeStruct((3, _B, _S, _DE), f32),
                   jax.ShapeDtypeStruct((3, _B, _S, 2 * _DE), f32)],
    )(h0, wts0, r2(ln2_g[0]), r2(ln2_b[0]),
      W1[0].transpose(1, 0, 2).reshape(_DE, _E * _HID),
      b1[0].reshape(1, _E * _HID),
      W2[0].reshape(_E * _HID, _DE), b2[0], expand,
      r2(ln1_g[1]), r2(ln1_b[1]), Wqkv[1], r2(bqkv[1]))

    h1, wts1 = attention(q1, kv1, x1, 1)

    tok3 = pl.BlockSpec((3, 1, _TS, _DE), lambda bb, si: (0, bb, si, 0))
    wspec3 = pl.BlockSpec((3, 1, _E, _TS), lambda bb, si: (0, bb, 0, si))
    out = pl.pallas_call(
        _fin_kernel,
        grid=(_B, nst),
        in_specs=[tok3, wspec3,
                  _full((1, _DE)), _full((1, _DE)),
                  _full((_DE, _E * _HID)), _full((1, _E * _HID)),
                  _full((_E * _HID, _DE)), _full((_E, _DE)),
                  _full((_E, _E * _HID)),
                  _full((_D, _D)), _full((1, _D)),
                  _full((_D, _C)), _full((1, _C))],
        out_specs=pl.BlockSpec((1, _TS, _C), lambda bb, si: (bb, si, 0)),
        out_shape=jax.ShapeDtypeStruct((_B, _S, _C), f32),
    )(h1, wts1, r2(ln2_g[1]), r2(ln2_b[1]),
      W1[1].transpose(1, 0, 2).reshape(_DE, _E * _HID),
      b1[1].reshape(1, _E * _HID),
      W2[1].reshape(_E * _HID, _DE), b2[1], expand,
      Wp1, r2(bp1), Wh, r2(bh))
    return out


# speedup vs baseline: 1.0799x; 1.0093x over previous
import functools
import math

import jax
import jax.numpy as jnp
from jax import lax
from jax.experimental import pallas as pl
from jax.experimental.pallas import tpu as pltpu
from jax.experimental.pallas import tpu_sc as plsc

_B, _S = 2, 2048
_DE = 128
_H = 4
_DH = _DE // _H
_E = 6
_HID = 128
_D = 3 * _DE
_C = 6

_TS = 512
_QB = 1024

_NEG = -1e30


def _f32dot(a, b):
    return jnp.dot(a, b, preferred_element_type=jnp.float32)


def _ln_block(x, g, b):
    m = jnp.mean(x, axis=-1, keepdims=True)
    d = x - m
    var = jnp.mean(d * d, axis=-1, keepdims=True)
    return d * jax.lax.rsqrt(var + 1e-5) * g + b


def _qkv_of(x, g, b, wqkv, bqkv):
    y = _ln_block(x, g, b)
    return _f32dot(y, wqkv) + bqkv


def _inproj_kernel(a_ref, t_ref, v_ref, wa, ba, wt, bt, wv, bv,
                   g1, b1, wqkv, bqkv, x_ref, q_ref, kv_ref):
    ins = ((a_ref, wa, ba), (t_ref, wt, bt), (v_ref, wv, bv))
    for m, (r, w, bb) in enumerate(ins):
        x = _f32dot(r[0], w[...]) + bb[...]
        x_ref[m, 0] = x
        qkv = _qkv_of(x, g1[...], b1[...], wqkv[...], bqkv[...])
        q_ref[m, 0] = qkv[:, :_DE]
        kv_ref[m, 0] = qkv[:, _DE:]


def _attn_kernel(q_ref, kv_ref, x_ref, wo, bo, g2, b2, wr_ref, br_ref,
                 h_ref, lg_ref):
    q_all = q_ref[0, 0]
    kv = kv_ref[0, 0]
    c = 1.4426950408889634 / math.sqrt(float(_DH))
    outs = []
    for h in range(_H):
        lo = h * _DH
        q = (q_all[:, lo:lo + _DH] * c).astype(jnp.bfloat16)
        k = kv[:, lo:lo + _DH].astype(jnp.bfloat16)
        v = kv[:, _DE + lo:_DE + lo + _DH]
        s = jax.lax.dot_general(q, k, (((1,), (1,)), ((), ())),
                                preferred_element_type=jnp.float32)
        p = jnp.exp2(s.astype(jnp.bfloat16))
        r = 1.0 / jnp.sum(p.astype(jnp.float32), axis=-1, keepdims=True)
        outs.append(jnp.dot(p, v.astype(jnp.bfloat16),
                            preferred_element_type=jnp.float32) * r)
    o = jnp.concatenate(outs, axis=-1)
    hh = x_ref[0, 0] + _f32dot(o, wo[...]) + bo[...]
    h_ref[0, 0] = hh
    z = _ln_block(hh, g2[...], b2[...])
    lt = jax.lax.dot_general(wr_ref[0], z, (((0,), (1,)), ((), ())),
                             preferred_element_type=jnp.float32)
    lg_ref[0, 0] = lt + br_ref[0]


def _moe_block(h, g2, b2, wts_t, w1all, b1all, w2all, b2mat, expand):
    z = _ln_block(h, g2, b2)
    hidden = jax.nn.gelu(_f32dot(z, w1all[...]) + b1all[...])
    wexp = jax.lax.dot_general(wts_t, expand[...], (((0,), (0,)), ((), ())),
                               preferred_element_type=jnp.float32)
    b2term = jax.lax.dot_general(wts_t, b2mat[...], (((0,), (0,)), ((), ())),
                                 preferred_element_type=jnp.float32)
    return h + _f32dot(wexp * hidden, w2all[...]) + b2term


def _sc_routing(logits):
    ginfo = plsc.get_sparse_core_info()
    nw = ginfo.num_cores * ginfo.num_subcores
    lanes = ginfo.num_lanes
    g_, e_, s_ = logits.shape
    chunk = s_ // nw

    mesh = plsc.VectorSubcoreMesh(core_axis_name="c", subcore_axis_name="s")

    @functools.partial(
        pl.kernel, mesh=mesh,
        out_type=jax.ShapeDtypeStruct((g_, e_, s_), jnp.float32),
        scratch_types=[pltpu.VMEM((2, e_, chunk), jnp.float32),
                       pltpu.VMEM((e_, chunk), jnp.float32),
                       pltpu.SemaphoreType.DMA,
                       pltpu.SemaphoreType.DMA,
                       pltpu.SemaphoreType.DMA],
    )
    def k(lg_hbm, out_hbm, lg_v, wt_v, sin0, sin1, sem):
        wid = lax.axis_index("s") * ginfo.num_cores + lax.axis_index("c")
        base = wid * chunk
        sin = (sin0, sin1)

        def fire_in(g):
            return [pltpu.async_copy(lg_hbm.at[g, e, pl.ds(base, chunk)],
                                     lg_v.at[g % 2, e], sin[g % 2])
                    for e in range(e_)]

        pending = fire_in(0)
        for g in range(g_):
            nxt = fire_in(g + 1) if g + 1 < g_ else []
            for cp in pending:
                cp.wait()
            pending = nxt
            buf = g % 2
            for cidx in range(chunk // lanes):
                sl = pl.ds(cidx * lanes, lanes)
                l = [lg_v[buf, e, sl] for e in range(e_)]
                m1 = l[0]
                for e in range(1, e_):
                    m1 = jnp.maximum(m1, l[e])
                taken = jnp.zeros_like(m1)
                msk1 = []
                for e in range(e_):
                    hit = jnp.where(l[e] == m1, 1.0 - taken, 0.0)
                    msk1.append(hit)
                    taken = taken + hit
                rest = [jnp.where(msk1[e] > 0.5, _NEG, l[e])
                        for e in range(e_)]
                m2 = rest[0]
                for e in range(1, e_):
                    m2 = jnp.maximum(m2, rest[e])
                g1w = 1.0 / (1.0 + jnp.exp(m2 - m1))
                g2w = 1.0 - g1w
                taken2 = jnp.zeros_like(m1)
                for e in range(e_):
                    hit2 = jnp.where(rest[e] == m2, 1.0 - taken2, 0.0)
                    taken2 = taken2 + hit2
                    wt_v[e, sl] = msk1[e] * g1w + hit2 * g2w
            ops = [pltpu.async_copy(wt_v.at[e],
                                    out_hbm.at[g, e, pl.ds(base, chunk)],
                                    sem) for e in range(e_)]
            for cp in ops:
                cp.wait()

    return k(logits)


def _mid_kernel(h_ref, wts_ref, g2, b2,
                w1s, b1s, w2s, b2s, expand, g1n, b1n, wqkvn, bqkvn,
                xn_ref, qn_ref, kvn_ref):
    h = h_ref[0, 0]
    acc = _moe_block(h, g2[...], b2[...], wts_ref[0, 0],
                     w1s, b1s, w2s, b2s, expand)
    xn_ref[0, 0] = acc
    qkv = _qkv_of(acc, g1n[...], b1n[...], wqkvn[...], bqkvn[...])
    qn_ref[0, 0] = qkv[:, :_DE]
    kvn_ref[0, 0] = qkv[:, _DE:]


def _fin_kernel(h_ref, wts_ref, g2, b2,
                w1s, b1s, w2s, b2s, expand, wp1, bp1, wh, bh, o_ref):
    parts = []
    for m in range(3):
        parts.append(_moe_block(h_ref[m, 0], g2[...], b2[...], wts_ref[m, 0],
                                w1s, b1s, w2s, b2s, expand))
    fused = jnp.concatenate(parts, axis=-1)
    hid = jnp.maximum(_f32dot(fused, wp1[...]) + bp1[...], 0.0)
    o_ref[0] = _f32dot(hid, wh[...]) + bh[...]


def _full(shape):
    n = len(shape)
    return pl.BlockSpec(shape, lambda *args: (0,) * n)


def kernel(a, t, v, Wa, ba, Wt, bt, Wv, bv, ln1_g, ln1_b, Wqkv, bqkv, Wo, bo,
           ln2_g, ln2_b, Wr, br, W1, b1, W2, b2, Wp1, bp1, Wh, bh):
    f32 = jnp.float32
    r2 = lambda x: x.reshape(1, -1)
    expand = jnp.kron(jnp.eye(_E, dtype=f32), jnp.ones((1, _HID), f32))

    nst = _S // _TS
    nqb = _S // _QB

    tok = lambda w: pl.BlockSpec((1, _TS, w), lambda bb, si: (bb, si, 0))
    qkv_outspecs = [
        pl.BlockSpec((3, 1, _TS, _DE), lambda bb, si: (0, bb, si, 0)),
        pl.BlockSpec((3, 1, _TS, _DE), lambda bb, si: (0, bb, si, 0)),
        pl.BlockSpec((3, 1, _TS, 2 * _DE), lambda bb, si: (0, bb, si, 0))]
    qkv_outshapes = [jax.ShapeDtypeStruct((3, _B, _S, _DE), f32),
                     jax.ShapeDtypeStruct((3, _B, _S, _DE), f32),
                     jax.ShapeDtypeStruct((3, _B, _S, 2 * _DE), f32)]
    x0, q0, kv0 = pl.pallas_call(
        _inproj_kernel,
        grid=(_B, nst),
        in_specs=[tok(a.shape[-1]), tok(t.shape[-1]), tok(v.shape[-1])]
                 + [_full(s) for s in ((Wa.shape), (1, _DE), (Wt.shape), (1, _DE),
                                       (Wv.shape), (1, _DE), (1, _DE), (1, _DE),
                                       (_DE, 3 * _DE), (1, 3 * _DE))],
        out_specs=qkv_outspecs,
        out_shape=qkv_outshapes,
    )(a, t, v, Wa, r2(ba), Wt, r2(bt), Wv, r2(bv),
      r2(ln1_g[0]), r2(ln1_b[0]), Wqkv[0], r2(bqkv[0]))

    tokq = pl.BlockSpec((1, 1, _QB, _DE), lambda m, bb, si: (m, bb, si, 0))

    def attention(q, kv, x, l):
        h, lg = pl.pallas_call(
            _attn_kernel,
            grid=(3, _B, nqb),
            in_specs=[tokq,
                      pl.BlockSpec((1, 1, _S, 2 * _DE), lambda m, bb, si: (m, bb, 0, 0)),
                      tokq,
                      _full((_DE, _DE)), _full((1, _DE)),
                      _full((1, _DE)), _full((1, _DE)),
                      pl.BlockSpec((1, _DE, _E), lambda m, bb, si: (m, 0, 0)),
                      pl.BlockSpec((1, _E, 1), lambda m, bb, si: (m, 0, 0))],
            out_specs=[tokq,
                       pl.BlockSpec((1, 1, _E, _QB), lambda m, bb, si: (m, bb, 0, si))],
            out_shape=[jax.ShapeDtypeStruct((3, _B, _S, _DE), f32),
                       jax.ShapeDtypeStruct((3, _B, _E, _S), f32)],
        )(q, kv, x, Wo[l], r2(bo[l]), r2(ln2_g[l]), r2(ln2_b[l]),
          Wr[l], br[l].reshape(3, _E, 1))
        wts = _sc_routing(lg.reshape(3 * _B, _E, _S)).reshape(3, _B, _E, _S)
        return h, wts

    h0, wts0 = attention(q0, kv0, x0, 0)

    tokde = pl.BlockSpec((1, 1, _TS, _DE), lambda m, bb, si: (m, bb, si, 0))
    wspec = pl.BlockSpec((1, 1, _E, _TS), lambda m, bb, si: (m, bb, 0, si))
    x1, q1, kv1 = pl.pallas_call(
        _mid_kernel,
        grid=(3, _B, nst),
        in_specs=[tokde, wspec,
                  _full((1, _DE)), _full((1, _DE)),
                  _full((_DE, _E * _HID)), _full((1, _E * _HID)),
                  _full((_E * _HID, _DE)), _full((_E, _DE)),
                  _full((_E, _E * _HID)),
                  _full((1, _DE)), _full((1, _DE)),
                  _full((_DE, 3 * _DE)), _full((1, 3 * _DE))],
        out_specs=[tokde, tokde,
                   pl.BlockSpec((1, 1, _TS, 2 * _DE), lambda m, bb, si: (m, bb, si, 0))],
        out_shape=[jax.ShapeDtypeStruct((3, _B, _S, _DE), f32),
                   jax.ShapeDtypeStruct((3, _B, _S, _DE), f32),
                   jax.ShapeDtypeStruct((3, _B, _S, 2 * _DE), f32)],
    )(h0, wts0, r2(ln2_g[0]), r2(ln2_b[0]),
      W1[0].transpose(1, 0, 2).reshape(_DE, _E * _HID),
      b1[0].reshape(1, _E * _HID),
      W2[0].reshape(_E * _HID, _DE), b2[0], expand,
      r2(ln1_g[1]), r2(ln1_b[1]), Wqkv[1], r2(bqkv[1]))

    h1, wts1 = attention(q1, kv1, x1, 1)

    tok3 = pl.BlockSpec((3, 1, _TS, _DE), lambda bb, si: (0, bb, si, 0))
    wspec3 = pl.BlockSpec((3, 1, _E, _TS), lambda bb, si: (0, bb, 0, si))
    out = pl.pallas_call(
        _fin_kernel,
        grid=(_B, nst),
        in_specs=[tok3, wspec3,
                  _full((1, _DE)), _full((1, _DE)),
                  _full((_DE, _E * _HID)), _full((1, _E * _HID)),
                  _full((_E * _HID, _DE)), _full((_E, _DE)),
                  _full((_E, _E * _HID)),
                  _full((_D, _D)), _full((1, _D)),
                  _full((_D, _C)), _full((1, _C))],
        out_specs=pl.BlockSpec((1, _TS, _C), lambda bb, si: (bb, si, 0)),
        out_shape=jax.ShapeDtypeStruct((_B, _S, _C), f32),
    )(h1, wts1, r2(ln2_g[1]), r2(ln2_b[1]),
      W1[1].transpose(1, 0, 2).reshape(_DE, _E * _HID),
      b1[1].reshape(1, _E * _HID),
      W2[1].reshape(_E * _HID, _DE), b2[1], expand,
      Wp1, r2(bp1), Wh, r2(bh))
    return out
